# Initial kernel scaffold; baseline (speedup 1.0000x reference)
import jax
import jax.numpy as jnp
from jax.experimental import pallas as pl


def kernel(x, edge_index, W1, b1, W2, b2, W3, b3, Wl, bl):
    raise NotImplementedError("write your pallas kernel here")


# speedup vs baseline: 29.4338x; 29.4338x over previous
import functools

import jax
import jax.numpy as jnp
from jax import lax
from jax.experimental import pallas as pl
from jax.experimental.pallas import tpu as pltpu
from jax.experimental.pallas import tpu_sc as plsc

N = 100000
E = 3200000
L = 16
NC = 2
NS = 16

BLK = 1024
N_TAB = 100352
NBLK = N_TAB // BLK
ROWS_PER_TILE = N_TAB // NS

GROUP = 128
GROUPS_PER_TILE = 784
EG = NC * NS * GROUPS_PER_TILE
E_PAD = EG * GROUP

GPC = 4
CHUNKS = GROUPS_PER_TILE // GPC

GPC_D = 8
CHUNKS_D = GROUPS_PER_TILE // GPC_D


@functools.cache
def _sc_mesh():
    return plsc.VectorSubcoreMesh(core_axis_name="c", subcore_axis_name="s",
                                  num_cores=NC, num_subcores=NS)


def _fill_flat(ref, words, val):
    def body(i, _):
        ref[pl.ds(i * L, L)] = jnp.full((L,), val, jnp.float32)
        return 0
    lax.fori_loop(0, words // L, body, 0)


@functools.cache
def _deg_kernel():
    @functools.partial(
        pl.kernel,
        out_type=jax.ShapeDtypeStruct((NC, N_TAB), jnp.float32),
        mesh=_sc_mesh(),
        compiler_params=pltpu.CompilerParams(use_tc_tiling_on_sc=False),
        scratch_types=[
            pltpu.VMEM((2, GPC_D, GROUP), jnp.int32),
            pltpu.VMEM((GROUP,), jnp.float32),
            pltpu.VMEM_SHARED((N_TAB,), jnp.float32),
            pltpu.SemaphoreType.DMA,
            pltpu.SemaphoreType.DMA,
        ],
    )
    def _deg_pass(dst_hbm, zeros_hbm, out_hbm, dst_v, ones_v, acc, s0, s1):
        cid = lax.axis_index("c")
        sid = lax.axis_index("s")
        wid = cid * NS + sid
        row0 = sid * ROWS_PER_TILE
        pltpu.sync_copy(zeros_hbm.at[pl.ds(row0, ROWS_PER_TILE)],
                        acc.at[pl.ds(row0, ROWS_PER_TILE)])
        _fill_flat(ones_v, GROUP, 1.0)
        plsc.subcore_barrier()
        gbase = wid * GROUPS_PER_TILE
        ssem = (s0, s1)

        def load(buf, cix):
            pltpu.sync_copy(dst_hbm.at[pl.ds(gbase + cix * GPC_D, GPC_D)],
                            dst_v.at[buf])

        def fire(buf):
            for j in range(GPC_D):
                pltpu.async_copy(ones_v, acc.at[dst_v.at[buf, j]],
                                 ssem[buf], add=True)

        def drain(buf):
            for j in range(GPC_D):
                pltpu.make_async_copy(ones_v, acc.at[dst_v.at[buf, j]],
                                      ssem[buf]).wait()

        load(0, 0)
        fire(0)
        load(1, 1)

        def pair(t, _):
            c0 = 2 * t + 1
            fire(1)
            drain(0)
            load(0, c0 + 1)
            fire(0)
            drain(1)
            load(1, c0 + 2)
            return 0

        lax.fori_loop(0, CHUNKS_D // 2 - 1, pair, 0)
        fire(1)
        drain(0)
        drain(1)
        plsc.subcore_barrier()
        pltpu.sync_copy(acc.at[pl.ds(row0, ROWS_PER_TILE)],
                        out_hbm.at[cid, pl.ds(row0, ROWS_PER_TILE)])

    return _deg_pass


@functools.cache
def _edge_kernel():
    @functools.partial(
        pl.kernel,
        out_type=jax.ShapeDtypeStruct((NC, N_TAB, L), jnp.float32),
        mesh=_sc_mesh(),
        compiler_params=pltpu.CompilerParams(use_tc_tiling_on_sc=False),
        scratch_types=[
            pltpu.VMEM((3, GPC, GROUP), jnp.int32),
            pltpu.VMEM((3, GPC, GROUP), jnp.int32),
            pltpu.VMEM((3, GPC, GROUP, L), jnp.float32),
            pltpu.VMEM_SHARED((N_TAB, L), jnp.float32),
            pltpu.SemaphoreType.DMA,
            pltpu.SemaphoreType.DMA,
            pltpu.SemaphoreType.DMA,
            pltpu.SemaphoreType.DMA,
            pltpu.SemaphoreType.DMA,
            pltpu.SemaphoreType.DMA,
        ],
    )
    def _edge_pass(src_hbm, dst_hbm, tab_hbm, zeros_hbm, out_hbm,
                   src_v, dst_v, rows_v, acc, g0, g1, g2, s0, s1, s2):
        cid = lax.axis_index("c")
        sid = lax.axis_index("s")
        wid = cid * NS + sid
        row0 = sid * ROWS_PER_TILE
        pltpu.sync_copy(zeros_hbm.at[pl.ds(row0, ROWS_PER_TILE)],
                        acc.at[pl.ds(row0, ROWS_PER_TILE)])
        plsc.subcore_barrier()
        gbase = wid * GROUPS_PER_TILE
        gsem = (g0, g1, g2)
        ssem = (s0, s1, s2)

        def load(buf, cix):
            gb = gbase + cix * GPC
            pltpu.sync_copy(src_hbm.at[pl.ds(gb, GPC)], src_v.at[buf])
            pltpu.sync_copy(dst_hbm.at[pl.ds(gb, GPC)], dst_v.at[buf])

        def fire_g(buf):
            for j in range(GPC):
                pltpu.async_copy(tab_hbm.at[src_v.at[buf, j]],
                                 rows_v.at[buf, j], gsem[buf])

        def wait_g(buf):
            for j in range(GPC):
                pltpu.make_async_copy(tab_hbm.at[src_v.at[buf, j]],
                                      rows_v.at[buf, j], gsem[buf]).wait()

        def fire_s(buf):
            for j in range(GPC):
                pltpu.async_copy(rows_v.at[buf, j], acc.at[dst_v.at[buf, j]],
                                 ssem[buf], add=True)

        def wait_s(buf):
            for j in range(GPC):
                pltpu.make_async_copy(rows_v.at[buf, j],
                                      acc.at[dst_v.at[buf, j]],
                                      ssem[buf]).wait()

        load(0, 0)
        fire_g(0)
        load(1, 1)
        fire_g(1)
        wait_g(0)
        fire_s(0)
        load(2, 2)
        fire_g(2)
        wait_g(1)
        fire_s(1)
        wait_s(0)
        load(0, 3)
        fire_g(0)

        def trip(t, _):
            c = 3 * t + 2
            wait_g(2)
            fire_s(2)
            wait_s(1)
            load(1, c + 2)
            fire_g(1)
            wait_g(0)
            fire_s(0)
            wait_s(2)
            load(2, c + 3)
            fire_g(2)
            wait_g(1)
            fire_s(1)
            wait_s(0)
            load(0, c + 4)
            fire_g(0)
            return 0

        lax.fori_loop(0, (CHUNKS - 4) // 3, trip, 0)
        wait_g(2)
        fire_s(2)
        wait_s(1)
        wait_g(0)
        fire_s(0)
        wait_s(2)
        wait_s(0)
        plsc.subcore_barrier()
        pltpu.sync_copy(acc.at[pl.ds(row0, ROWS_PER_TILE)],
                        out_hbm.at[cid, pl.ds(row0, ROWS_PER_TILE)])

    return _edge_pass


def _t0_body(acc_ref, x_ref, w1_ref, g1_ref, dinv_ref):
    a = acc_ref[...]
    deg = a[0] + a[1] + 1.0
    dinv = lax.rsqrt(deg)
    hw = jnp.dot(x_ref[...], w1_ref[...], preferred_element_type=jnp.float32)
    g1_ref[...] = hw * dinv
    dinv_ref[...] = dinv


def _layer_body(k_in, k_out, refs):
    accs = refs[:k_in]
    gs = refs[k_in:2 * k_in]
    dinv_ref, b_ref, w_ref = refs[2 * k_in:2 * k_in + 3]
    outs = refs[2 * k_in + 3:]
    dinv = dinv_ref[...]
    b = b_ref[...]
    hs = []
    for k in range(k_in):
        a = accs[k][...]
        s = (a[0] + a[1] + gs[k][...]) * dinv + b[:, k * L:(k + 1) * L]
        hs.append(jnp.tanh(s))
    h = jnp.concatenate(hs, axis=1) if k_in > 1 else hs[0]
    hw = jnp.dot(h, w_ref[...], preferred_element_type=jnp.float32)
    g = hw * dinv
    for k in range(k_out):
        outs[k][...] = g[:, k * L:(k + 1) * L]


def _t1_body(*refs):
    _layer_body(1, 2, refs)


def _t2_body(*refs):
    _layer_body(2, 4, refs)


def _t3_body(a0, a1, a2, a3, g0, g1, g2, g3,
             dinv_ref, b_ref, wl_ref, bl_ref, out_ref, m_ref):
    i = pl.program_id(0)
    dinv = dinv_ref[...]
    b = b_ref[...]
    hs = []
    for k, (ar, gr) in enumerate(zip((a0, a1, a2, a3), (g0, g1, g2, g3))):
        a = ar[...]
        s = (a[0] + a[1] + gr[...]) * dinv + b[:, k * L:(k + 1) * L]
        hs.append(jnp.tanh(s))
    h = jnp.concatenate(hs, axis=1)
    rid = i * BLK + lax.broadcasted_iota(jnp.int32, (BLK, 1), 0)
    h = jnp.where(rid < N, h, -2.0)

    @pl.when(i == 0)
    def _():
        m_ref[...] = h

    @pl.when(i > 0)
    def _():
        m_ref[...] = jnp.maximum(m_ref[...], h)

    @pl.when(i == NBLK - 1)
    def _():
        m = jnp.max(m_ref[...], axis=0, keepdims=True)
        out_ref[...] = (jnp.dot(m, wl_ref[...], preferred_element_type=jnp.float32)
                        + bl_ref[...])


def _acc_spec():
    return pl.BlockSpec((2, BLK, L), lambda i: (0, i, 0))


def _tab_spec():
    return pl.BlockSpec((BLK, L), lambda i: (i, 0))


def _full_spec(shape):
    return pl.BlockSpec(shape, lambda i: tuple(0 for _ in shape))


def kernel(x, edge_index, W1, b1, W2, b2, W3, b3, Wl, bl):
    f32 = jnp.float32
    ei = edge_index.astype(jnp.int32)
    pad = jnp.full((E_PAD - E,), N, jnp.int32)
    srcg = jnp.concatenate([ei[0], pad]).reshape(EG, GROUP)
    dstg = jnp.concatenate([ei[1], pad]).reshape(EG, GROUP)
    x_pad = jnp.pad(x, ((0, N_TAB - N), (0, 0)))
    zeros_tab = jnp.zeros((N_TAB, L), f32)
    zeros_deg = jnp.zeros((N_TAB,), f32)

    deg_acc = _deg_kernel()(dstg, zeros_deg).reshape(NC, N_TAB, 1)

    g1, dinv = pl.pallas_call(
        _t0_body,
        grid=(NBLK,),
        in_specs=[pl.BlockSpec((2, BLK, 1), lambda i: (0, i, 0)),
                  pl.BlockSpec((BLK, 4), lambda i: (i, 0)),
                  _full_spec((4, L))],
        out_specs=[_tab_spec(), pl.BlockSpec((BLK, 1), lambda i: (i, 0))],
        out_shape=[jax.ShapeDtypeStruct((N_TAB, L), f32),
                   jax.ShapeDtypeStruct((N_TAB, 1), f32)],
    )(deg_acc, x_pad, W1)

    acc1 = _edge_kernel()(srcg, dstg, g1, zeros_tab)

    dinv_spec = pl.BlockSpec((BLK, 1), lambda i: (i, 0))
    g2a, g2b = pl.pallas_call(
        _t1_body,
        grid=(NBLK,),
        in_specs=[_acc_spec(), _tab_spec(), dinv_spec,
                  _full_spec((1, 16)), _full_spec((16, 32))],
        out_specs=[_tab_spec(), _tab_spec()],
        out_shape=[jax.ShapeDtypeStruct((N_TAB, L), f32)] * 2,
    )(acc1, g1, dinv, b1.reshape(1, 16), W2)

    acc2a = _edge_kernel()(srcg, dstg, g2a, zeros_tab)
    acc2b = _edge_kernel()(srcg, dstg, g2b, zeros_tab)

    g3 = pl.pallas_call(
        _t2_body,
        grid=(NBLK,),
        in_specs=[_acc_spec(), _acc_spec(), _tab_spec(), _tab_spec(), dinv_spec,
                  _full_spec((1, 32)), _full_spec((32, 64))],
        out_specs=[_tab_spec()] * 4,
        out_shape=[jax.ShapeDtypeStruct((N_TAB, L), f32)] * 4,
    )(acc2a, acc2b, g2a, g2b, dinv, b2.reshape(1, 32), W3)

    accs3 = [_edge_kernel()(srcg, dstg, g, zeros_tab) for g in g3]

    out = pl.pallas_call(
        _t3_body,
        grid=(NBLK,),
        in_specs=[_acc_spec()] * 4 + [_tab_spec()] * 4
                 + [dinv_spec, _full_spec((1, 64)), _full_spec((64, 1)),
                    _full_spec((1, 1))],
        out_specs=pl.BlockSpec((1, 1), lambda i: (0, 0)),
        out_shape=jax.ShapeDtypeStruct((1, 1), f32),
        scratch_shapes=[pltpu.VMEM((BLK, 64), f32)],
    )(*accs3, *g3, dinv, b3.reshape(1, 64), Wl, bl.reshape(1, 1))
    return out
